# Initial kernel scaffold; baseline (speedup 1.0000x reference)
import jax
import jax.numpy as jnp
from jax.experimental import pallas as pl


def kernel(query, cases, case_labels, fa_weight, ca_weight, ca_bias, cls_weight, cls_bias):
    raise NotImplementedError("write your pallas kernel here")


# speedup vs baseline: 2.0274x; 2.0274x over previous
import functools

import jax
import jax.numpy as jnp
from jax import lax
from jax.experimental import pallas as pl

B = 32
C = 50000
F = 32
L = 10
K = 32


def _phase_a_body(casesT_ref, qT_ref, faw_ref, caw_ref, cab_ref, fa_ref, v_ref):
    ib = pl.program_id(0)
    casesT = casesT_ref[...]
    qT = qT_ref[...]
    lane = lax.broadcasted_iota(jnp.int32, (F, B), 1)
    qcol = jnp.sum(jnp.where(lane == ib, qT, jnp.float32(0)), axis=1,
                   keepdims=True)
    w_fa = jnp.maximum(faw_ref[...], jnp.float32(0))
    d = jnp.abs(qcol - casesT)
    fa_b = jnp.exp(jnp.negative(d) * w_fa)
    fa_ref[0] = fa_b

    w_ca = jnp.maximum(caw_ref[...], jnp.float32(0)).astype(jnp.bfloat16)
    ca = lax.dot_general(w_ca, fa_b, (((1,), (0,)), ((), ())),
                         preferred_element_type=jnp.float32)
    ca = ca + cab_ref[...]
    one = jnp.float32(1)
    v_ref[0] = one / (jnp.exp(jnp.negative(ca)) + one)


def _phase_b_body(v_ref, labels_ref, clsT_ref, clsb_ref,
                  mv_ref, out_ref, pred_ref):
    v = v_ref[...]
    u = lax.bitcast_convert_type(v, jnp.int32)
    col = lax.broadcasted_iota(jnp.int32, (B, C), 1)

    def count_gt(x):
        return jnp.sum(jnp.where(u > x, jnp.int32(1), jnp.int32(0)),
                       axis=1, keepdims=True)

    def val_step(_, carry):
        lo, hi = carry
        mid = lo + (hi - lo) // 2
        ge = count_gt(mid) >= K
        return jnp.where(ge, mid + 1, lo), jnp.where(ge, hi, mid)

    lo0 = jnp.zeros((B, 1), jnp.int32)
    hi0 = jnp.full((B, 1), jnp.int32(0x7F800000))
    lo, hi = lax.fori_loop(0, 31, val_step, (lo0, hi0))
    t = lo

    gt = u > t
    eq = u == t
    n_gt = jnp.sum(jnp.where(gt, jnp.int32(1), jnp.int32(0)), axis=1,
                   keepdims=True)
    m = jnp.int32(K) - n_gt

    def count_eq_le(j):
        sel = jnp.logical_and(eq, col <= j)
        return jnp.sum(jnp.where(sel, jnp.int32(1), jnp.int32(0)), axis=1,
                       keepdims=True)

    def idx_step(_, carry):
        jlo, jhi = carry
        jmid = jlo + (jhi - jlo) // 2
        enough = count_eq_le(jmid) >= m
        return jnp.where(enough, jlo, jmid + 1), jnp.where(enough, jmid, jhi)

    jlo0 = jnp.zeros((B, 1), jnp.int32)
    jhi0 = jnp.full((B, 1), jnp.int32(C - 1))
    jlo, jhi = lax.fori_loop(0, 16, idx_step, (jlo0, jhi0))
    cutoff = jnp.where(m > 0, jlo, jnp.int32(-1))

    sel = jnp.logical_or(gt, jnp.logical_and(eq, col <= cutoff))
    mv = jnp.where(sel, v, jnp.float32(0))
    mv_ref[...] = mv

    lab = labels_ref[...]
    liota = lax.broadcasted_iota(jnp.int32, (L, C), 0)
    wt = jnp.where(lab == liota,
                   jnp.maximum(clsT_ref[...], jnp.float32(0)).astype(jnp.bfloat16),
                   jnp.bfloat16(0))
    vhi = mv.astype(jnp.bfloat16)
    vlo = (mv - vhi.astype(jnp.float32)).astype(jnp.bfloat16)
    nt = (((1,), (1,)), ((), ()))
    out = (lax.dot_general(vhi, wt, nt, preferred_element_type=jnp.float32)
           + lax.dot_general(vlo, wt, nt, preferred_element_type=jnp.float32))
    out = out + clsb_ref[...]
    out_ref[...] = out

    mx = jnp.max(out, axis=1, keepdims=True)
    li = lax.broadcasted_iota(jnp.int32, (B, L), 1)
    pred = jnp.min(jnp.where(out == mx, li, jnp.int32(C)), axis=1)
    pred_ref[...] = pred[None]


@jax.jit
def kernel(query, cases, case_labels, fa_weight, ca_weight, ca_bias,
           cls_weight, cls_bias):
    casesT = cases.T
    qT = query.T
    faw = fa_weight.reshape(F, 1)
    caw = ca_weight.reshape(1, F)
    cab = ca_bias.reshape(1, C)

    fa_bfc, v = pl.pallas_call(
        _phase_a_body,
        grid=(B,),
        in_specs=[
            pl.BlockSpec((F, C), lambda i: (0, 0)),
            pl.BlockSpec((F, B), lambda i: (0, 0)),
            pl.BlockSpec((F, 1), lambda i: (0, 0)),
            pl.BlockSpec((1, F), lambda i: (0, 0)),
            pl.BlockSpec((1, C), lambda i: (0, 0)),
        ],
        out_specs=[
            pl.BlockSpec((1, F, C), lambda i: (i, 0, 0)),
            pl.BlockSpec((1, 1, C), lambda i: (i, 0, 0)),
        ],
        out_shape=[
            jax.ShapeDtypeStruct((B, F, C), jnp.float32),
            jax.ShapeDtypeStruct((B, 1, C), jnp.float32),
        ],
    )(casesT, qT, faw, caw, cab)
    v = v.reshape(B, C)

    labels = case_labels.reshape(1, C)
    clsT = cls_weight.T
    clsb = cls_bias.reshape(1, L)

    mv, out, pred = pl.pallas_call(
        _phase_b_body,
        in_specs=[
            pl.BlockSpec((B, C), lambda: (0, 0)),
            pl.BlockSpec((1, C), lambda: (0, 0)),
            pl.BlockSpec((L, C), lambda: (0, 0)),
            pl.BlockSpec((1, L), lambda: (0, 0)),
        ],
        out_specs=[
            pl.BlockSpec((B, C), lambda: (0, 0)),
            pl.BlockSpec((B, L), lambda: (0, 0)),
            pl.BlockSpec((1, B), lambda: (0, 0)),
        ],
        out_shape=[
            jax.ShapeDtypeStruct((B, C), jnp.float32),
            jax.ShapeDtypeStruct((B, L), jnp.float32),
            jax.ShapeDtypeStruct((1, B), jnp.int32),
        ],
    )(v, labels, clsT, clsb)

    feature_activations = jnp.swapaxes(fa_bfc, 1, 2)
    return feature_activations, mv, out, pred.reshape(B)
